# baseline (speedup 1.0000x reference)
import functools

import jax
import jax.numpy as jnp
from jax import lax
from jax.experimental import pallas as pl
from jax.experimental.pallas import tpu as pltpu
from jax.experimental.pallas import tpu_sc as plsc

N = 10000
E = 320000
C = 128
HID = 128
OUT_C = 64
G = 64

KE = 128
NCH = E // KE
NC, NS = 2, 16
NW = NC * NS
EPW = NCH // NW
TAIL = NCH - EPW * NW
EMAIN = EPW * NW * KE
CSO = 200
NZO = N // CSO
NBUF = 2
NGRP = EPW // NBUF

BN = 1000
NB = N // BN


def _seg_sum_body(with_deg, *refs):
    if with_deg:
        (table, srcp, dstp, srct, dstt, zrows, ones_hbm, part, degpart,
         sidxbuf, d0, d1, r0, r1, gsem, ssem, isem, acc) = refs
    else:
        (table, srcp, dstp, srct, dstt, zrows, part,
         sidxbuf, d0, d1, r0, r1, gsem, ssem, isem, acc) = refs
    rows = (r0, r1)
    didx = (d0, d1)

    c = lax.axis_index("c")
    s = lax.axis_index("s")
    w = s * NC + c

    pltpu.sync_copy(srcp.at[w], sidxbuf)

    for t in range(4):
        j = s + NS * t

        @pl.when(j < NZO)
        def _(j=j):
            base = pl.multiple_of(j * CSO, 8)
            pltpu.sync_copy(zrows, acc.at[pl.ds(base, CSO)])

    plsc.subcore_barrier()

    @pl.loop(0, NGRP)
    def _(g):
        j0 = g * NBUF
        ids = []
        gds = []
        for b in range(NBUF):
            ids.append(pltpu.async_copy(dstp.at[w, j0 + b], didx[b], isem))
            gds.append(pltpu.async_copy(
                table.at[sidxbuf.at[j0 + b]], rows[b], gsem))
        sds = []
        for b in range(NBUF):
            ids[b].wait()
            gds[b].wait()
            sds.append(pltpu.async_copy(
                rows[b], acc.at[didx[b]], ssem, add=True))
        for b in range(NBUF):
            sds[b].wait()

    @pl.when(w < TAIL)
    def _():
        pltpu.sync_copy(srct.at[w], d0)
        pltpu.sync_copy(dstt.at[w], d1)
        pltpu.async_copy(table.at[d0], r0, gsem).wait()
        pltpu.sync_copy(r0, acc.at[d1], add=True)

    plsc.subcore_barrier()

    for t in range(4):
        j = s + NS * t

        @pl.when(j < NZO)
        def _(j=j):
            base = pl.multiple_of(j * CSO, 8)
            pltpu.sync_copy(acc.at[pl.ds(base, CSO)],
                            part.at[c, pl.ds(base, CSO)])

    if with_deg:
        plsc.subcore_barrier()

        pltpu.sync_copy(ones_hbm, r0)

        for t in range(4):
            j = s + NS * t

            @pl.when(j < NZO)
            def _(j=j):
                base = pl.multiple_of(j * CSO, 8)
                pltpu.sync_copy(zrows, acc.at[pl.ds(base, CSO)])

        plsc.subcore_barrier()

        @pl.loop(0, NGRP)
        def _(g):
            j0 = g * NBUF
            ids = []
            for b in range(NBUF):
                ids.append(pltpu.async_copy(dstp.at[w, j0 + b], didx[b], isem))
            sds = []
            for b in range(NBUF):
                ids[b].wait()
                sds.append(pltpu.async_copy(
                    r0, acc.at[didx[b]], ssem, add=True))
            for b in range(NBUF):
                sds[b].wait()

        @pl.when(w < TAIL)
        def _():
            pltpu.sync_copy(dstt.at[w], d1)
            pltpu.sync_copy(r0, acc.at[d1], add=True)

        plsc.subcore_barrier()

        for t in range(4):
            j = s + NS * t

            @pl.when(j < NZO)
            def _(j=j):
                base = pl.multiple_of(j * CSO, 8)
                pltpu.sync_copy(acc.at[pl.ds(base, CSO)],
                                degpart.at[c, pl.ds(base, CSO)])


_SC_MESH = plsc.VectorSubcoreMesh(core_axis_name="c", subcore_axis_name="s")


def _sc_scratch():
    return [
        pltpu.VMEM((EPW, KE), jnp.int32),
        pltpu.VMEM((KE,), jnp.int32),
        pltpu.VMEM((KE,), jnp.int32),
        pltpu.VMEM((KE, C), jnp.float32),
        pltpu.VMEM((KE, C), jnp.float32),
        pltpu.SemaphoreType.DMA,
        pltpu.SemaphoreType.DMA,
        pltpu.SemaphoreType.DMA,
        pltpu.MemorySpace.VMEM_SHARED((N, C), jnp.float32),
    ]


_seg_sum_deg = pl.kernel(
    functools.partial(_seg_sum_body, True),
    out_type=(
        jax.ShapeDtypeStruct((NC, N, C), jnp.float32),
        jax.ShapeDtypeStruct((NC, N, C), jnp.float32),
    ),
    mesh=_SC_MESH,
    scratch_types=_sc_scratch(),
)

_seg_sum = pl.kernel(
    functools.partial(_seg_sum_body, False),
    out_type=jax.ShapeDtypeStruct((NC, N, C), jnp.float32),
    mesh=_SC_MESH,
    scratch_types=_sc_scratch(),
)


def _dot_t(a, b):
    return lax.dot_general(a, b, (((1,), (1,)), ((), ())),
                           preferred_element_type=jnp.float32,
                           precision=lax.Precision.HIGHEST)


def _layer_body(part, degp, x, wl, bl, wr, out):
    p = part[0] + part[1]
    deg = degp[0, :, 0] + degp[1, :, 0]
    aggn = p / jnp.clip(deg, 1.0, None)[:, None]
    h = _dot_t(aggn, wl[...]) + bl[...] + _dot_t(x[...], wr[...])
    out[...] = jnp.maximum(h, 0.0)


def _layer2_pool_body(part, degp, h0, wl, bl, wr, batch, wlin, blin, out,
                      pooled, cnt):
    i = pl.program_id(0)

    @pl.when(i == 0)
    def _():
        pooled[...] = jnp.zeros((G, HID), jnp.float32)
        cnt[...] = jnp.zeros((1, G), jnp.float32)

    p = part[0] + part[1]
    deg = degp[0, :, 0] + degp[1, :, 0]
    aggn = p / jnp.clip(deg, 1.0, None)[:, None]
    h1 = jnp.maximum(_dot_t(aggn, wl[...]) + bl[...] + _dot_t(h0[...], wr[...]),
                     0.0)
    bids = batch[0, 0, :]
    gids = lax.broadcasted_iota(jnp.int32, (G, BN), 0)
    m = (bids[None, :] == gids).astype(jnp.float32)
    pooled[...] += lax.dot_general(m, h1, (((1,), (0,)), ((), ())),
                                   preferred_element_type=jnp.float32,
                                   precision=lax.Precision.HIGHEST)
    cnt[...] += jnp.sum(m, axis=1)[None, :]

    @pl.when(i == NB - 1)
    def _():
        pn = pooled[...] / jnp.clip(cnt[0, :], 1.0, None)[:, None]
        out[...] = _dot_t(pn, wlin[...]) + blin[...]


def kernel(x, edge_index, batch, Wl0, bl0, Wr0, Wl1, bl1, Wr1, W_lin, b_lin):
    src = edge_index[0]
    dst = edge_index[1]
    srcp = src[:EMAIN].reshape(NW, EPW, KE)
    dstp = dst[:EMAIN].reshape(NW, EPW, KE)
    srct = src[EMAIN:].reshape(TAIL, KE)
    dstt = dst[EMAIN:].reshape(TAIL, KE)
    zrows = jnp.zeros((CSO, C), jnp.float32)
    ones_rows = jnp.ones((KE, C), jnp.float32)
    batch3d = batch.reshape(NB, 1, BN)
    bl0r = bl0.reshape(1, HID)
    bl1r = bl1.reshape(1, HID)
    blinr = b_lin.reshape(1, OUT_C)

    part0, degpart = _seg_sum_deg(x, srcp, dstp, srct, dstt, zrows, ones_rows)

    h0 = pl.pallas_call(
        _layer_body,
        grid=(NB,),
        in_specs=[
            pl.BlockSpec((NC, BN, C), lambda i: (0, i, 0)),
            pl.BlockSpec((NC, BN, C), lambda i: (0, i, 0)),
            pl.BlockSpec((BN, C), lambda i: (i, 0)),
            pl.BlockSpec((HID, C), lambda i: (0, 0)),
            pl.BlockSpec((1, HID), lambda i: (0, 0)),
            pl.BlockSpec((HID, C), lambda i: (0, 0)),
        ],
        out_specs=pl.BlockSpec((BN, HID), lambda i: (i, 0)),
        out_shape=jax.ShapeDtypeStruct((N, HID), jnp.float32),
    )(part0, degpart, x, Wl0, bl0r, Wr0)

    part1 = _seg_sum(h0, srcp, dstp, srct, dstt, zrows)

    out = pl.pallas_call(
        _layer2_pool_body,
        grid=(NB,),
        in_specs=[
            pl.BlockSpec((NC, BN, HID), lambda i: (0, i, 0)),
            pl.BlockSpec((NC, BN, C), lambda i: (0, i, 0)),
            pl.BlockSpec((BN, HID), lambda i: (i, 0)),
            pl.BlockSpec((HID, HID), lambda i: (0, 0)),
            pl.BlockSpec((1, HID), lambda i: (0, 0)),
            pl.BlockSpec((HID, HID), lambda i: (0, 0)),
            pl.BlockSpec((1, 1, BN), lambda i: (i, 0, 0)),
            pl.BlockSpec((OUT_C, HID), lambda i: (0, 0)),
            pl.BlockSpec((1, OUT_C), lambda i: (0, 0)),
        ],
        out_specs=pl.BlockSpec((G, OUT_C), lambda i: (0, 0)),
        out_shape=jax.ShapeDtypeStruct((G, OUT_C), jnp.float32),
        scratch_shapes=[
            pltpu.VMEM((G, HID), jnp.float32),
            pltpu.VMEM((1, G), jnp.float32),
        ],
    )(part1, degpart, h0, Wl1, bl1r, Wr1, batch3d, W_lin, blinr)

    return out


# speedup vs baseline: 1.0305x; 1.0305x over previous
import functools

import jax
import jax.numpy as jnp
from jax import lax
from jax.experimental import pallas as pl
from jax.experimental.pallas import tpu as pltpu
from jax.experimental.pallas import tpu_sc as plsc

N = 10000
E = 320000
C = 128
HID = 128
OUT_C = 64
G = 64

KE = 128
NCH = E // KE
NC, NS = 2, 16
NW = NC * NS
EPW = NCH // NW
TAIL = NCH - EPW * NW
EMAIN = EPW * NW * KE
CSO = 2000
NZO = N // CSO
TINIT = (NZO + NS - 1) // NS
NBUF = 3
NGRP = EPW // NBUF

BN = 1000
NB = N // BN


def _seg_sum_body(with_deg, *refs):
    if with_deg:
        (table, srcp, dstp, srct, dstt, zrows, ones_hbm, part, degpart,
         s0, s1, s2, d0, d1, d2, r0, r1, r2, gsem, ssem, isem, acc) = refs
    else:
        (table, srcp, dstp, srct, dstt, zrows, part,
         s0, s1, s2, d0, d1, d2, r0, r1, r2, gsem, ssem, isem, acc) = refs
    rows = (r0, r1, r2)
    didx = (d0, d1, d2)
    sidx = (s0, s1, s2)

    c = lax.axis_index("c")
    s = lax.axis_index("s")
    w = s * NC + c

    for t in range(TINIT):
        j = s + NS * t

        @pl.when(j < NZO)
        def _(j=j):
            base = pl.multiple_of(j * CSO, 8)
            pltpu.sync_copy(zrows, acc.at[pl.ds(base, CSO)])

    plsc.subcore_barrier()

    @pl.loop(0, NGRP)
    def _(g):
        j0 = g * NBUF
        isd = []
        ids = []
        for b in range(NBUF):
            isd.append(pltpu.async_copy(srcp.at[w, j0 + b], sidx[b], isem))
            ids.append(pltpu.async_copy(dstp.at[w, j0 + b], didx[b], isem))
        gds = []
        for b in range(NBUF):
            isd[b].wait()
            gds.append(pltpu.async_copy(
                table.at[sidx[b]], rows[b], gsem))
        sds = []
        for b in range(NBUF):
            ids[b].wait()
            gds[b].wait()
            sds.append(pltpu.async_copy(
                rows[b], acc.at[didx[b]], ssem, add=True))
        for b in range(NBUF):
            sds[b].wait()

    @pl.when(w < TAIL)
    def _():
        pltpu.sync_copy(srct.at[w], d0)
        pltpu.sync_copy(dstt.at[w], d1)
        pltpu.async_copy(table.at[d0], r0, gsem).wait()
        pltpu.sync_copy(r0, acc.at[d1], add=True)

    plsc.subcore_barrier()

    for t in range(TINIT):
        j = s + NS * t

        @pl.when(j < NZO)
        def _(j=j):
            base = pl.multiple_of(j * CSO, 8)
            pltpu.sync_copy(acc.at[pl.ds(base, CSO)],
                            part.at[c, pl.ds(base, CSO)])

    if with_deg:
        plsc.subcore_barrier()

        pltpu.sync_copy(ones_hbm, r0)

        for t in range(TINIT):
            j = s + NS * t

            @pl.when(j < NZO)
            def _(j=j):
                base = pl.multiple_of(j * CSO, 8)
                pltpu.sync_copy(zrows, acc.at[pl.ds(base, CSO)])

        plsc.subcore_barrier()

        @pl.loop(0, NGRP)
        def _(g):
            j0 = g * NBUF
            ids = []
            for b in range(NBUF):
                ids.append(pltpu.async_copy(dstp.at[w, j0 + b], didx[b], isem))
            sds = []
            for b in range(NBUF):
                ids[b].wait()
                sds.append(pltpu.async_copy(
                    r0, acc.at[didx[b]], ssem, add=True))
            for b in range(NBUF):
                sds[b].wait()

        @pl.when(w < TAIL)
        def _():
            pltpu.sync_copy(dstt.at[w], d1)
            pltpu.sync_copy(r0, acc.at[d1], add=True)

        plsc.subcore_barrier()

        for t in range(TINIT):
            j = s + NS * t

            @pl.when(j < NZO)
            def _(j=j):
                base = pl.multiple_of(j * CSO, 8)
                pltpu.sync_copy(acc.at[pl.ds(base, CSO)],
                                degpart.at[c, pl.ds(base, CSO)])


_SC_MESH = plsc.VectorSubcoreMesh(core_axis_name="c", subcore_axis_name="s")


def _sc_scratch():
    return [
        pltpu.VMEM((KE,), jnp.int32),
        pltpu.VMEM((KE,), jnp.int32),
        pltpu.VMEM((KE,), jnp.int32),
        pltpu.VMEM((KE,), jnp.int32),
        pltpu.VMEM((KE,), jnp.int32),
        pltpu.VMEM((KE,), jnp.int32),
        pltpu.VMEM((KE, C), jnp.float32),
        pltpu.VMEM((KE, C), jnp.float32),
        pltpu.VMEM((KE, C), jnp.float32),
        pltpu.SemaphoreType.DMA,
        pltpu.SemaphoreType.DMA,
        pltpu.SemaphoreType.DMA,
        pltpu.MemorySpace.VMEM_SHARED((N, C), jnp.float32),
    ]


_seg_sum_deg = pl.kernel(
    functools.partial(_seg_sum_body, True),
    out_type=(
        jax.ShapeDtypeStruct((NC, N, C), jnp.float32),
        jax.ShapeDtypeStruct((NC, N, C), jnp.float32),
    ),
    mesh=_SC_MESH,
    scratch_types=_sc_scratch(),
)

_seg_sum = pl.kernel(
    functools.partial(_seg_sum_body, False),
    out_type=jax.ShapeDtypeStruct((NC, N, C), jnp.float32),
    mesh=_SC_MESH,
    scratch_types=_sc_scratch(),
)


def _dot_t(a, b):
    return lax.dot_general(a, b, (((1,), (1,)), ((), ())),
                           preferred_element_type=jnp.float32,
                           precision=lax.Precision.HIGHEST)


def _layer_body(part, degp, x, wl, bl, wr, out):
    p = part[0] + part[1]
    deg = degp[0, :, 0] + degp[1, :, 0]
    aggn = p / jnp.clip(deg, 1.0, None)[:, None]
    h = _dot_t(aggn, wl[...]) + bl[...] + _dot_t(x[...], wr[...])
    out[...] = jnp.maximum(h, 0.0)


def _layer2_pool_body(part, degp, h0, wl, bl, wr, batch, wlin, blin, out,
                      pooled, cnt):
    i = pl.program_id(0)

    @pl.when(i == 0)
    def _():
        pooled[...] = jnp.zeros((G, HID), jnp.float32)
        cnt[...] = jnp.zeros((1, G), jnp.float32)

    p = part[0] + part[1]
    deg = degp[0, :, 0] + degp[1, :, 0]
    aggn = p / jnp.clip(deg, 1.0, None)[:, None]
    h1 = jnp.maximum(_dot_t(aggn, wl[...]) + bl[...] + _dot_t(h0[...], wr[...]),
                     0.0)
    bids = batch[0, 0, :]
    gids = lax.broadcasted_iota(jnp.int32, (G, BN), 0)
    m = (bids[None, :] == gids).astype(jnp.float32)
    pooled[...] += lax.dot_general(m, h1, (((1,), (0,)), ((), ())),
                                   preferred_element_type=jnp.float32,
                                   precision=lax.Precision.HIGHEST)
    cnt[...] += jnp.sum(m, axis=1)[None, :]

    @pl.when(i == NB - 1)
    def _():
        pn = pooled[...] / jnp.clip(cnt[0, :], 1.0, None)[:, None]
        out[...] = _dot_t(pn, wlin[...]) + blin[...]


def kernel(x, edge_index, batch, Wl0, bl0, Wr0, Wl1, bl1, Wr1, W_lin, b_lin):
    src = edge_index[0]
    dst = edge_index[1]
    srcp = src[:EMAIN].reshape(NW, EPW, KE)
    dstp = dst[:EMAIN].reshape(NW, EPW, KE)
    srct = src[EMAIN:].reshape(TAIL, KE)
    dstt = dst[EMAIN:].reshape(TAIL, KE)
    zrows = jnp.zeros((CSO, C), jnp.float32)
    ones_rows = jnp.ones((KE, C), jnp.float32)
    batch3d = batch.reshape(NB, 1, BN)
    bl0r = bl0.reshape(1, HID)
    bl1r = bl1.reshape(1, HID)
    blinr = b_lin.reshape(1, OUT_C)

    part0, degpart = _seg_sum_deg(x, srcp, dstp, srct, dstt, zrows, ones_rows)

    h0 = pl.pallas_call(
        _layer_body,
        grid=(NB,),
        in_specs=[
            pl.BlockSpec((NC, BN, C), lambda i: (0, i, 0)),
            pl.BlockSpec((NC, BN, C), lambda i: (0, i, 0)),
            pl.BlockSpec((BN, C), lambda i: (i, 0)),
            pl.BlockSpec((HID, C), lambda i: (0, 0)),
            pl.BlockSpec((1, HID), lambda i: (0, 0)),
            pl.BlockSpec((HID, C), lambda i: (0, 0)),
        ],
        out_specs=pl.BlockSpec((BN, HID), lambda i: (i, 0)),
        out_shape=jax.ShapeDtypeStruct((N, HID), jnp.float32),
    )(part0, degpart, x, Wl0, bl0r, Wr0)

    part1 = _seg_sum(h0, srcp, dstp, srct, dstt, zrows)

    out = pl.pallas_call(
        _layer2_pool_body,
        grid=(NB,),
        in_specs=[
            pl.BlockSpec((NC, BN, HID), lambda i: (0, i, 0)),
            pl.BlockSpec((NC, BN, C), lambda i: (0, i, 0)),
            pl.BlockSpec((BN, HID), lambda i: (i, 0)),
            pl.BlockSpec((HID, HID), lambda i: (0, 0)),
            pl.BlockSpec((1, HID), lambda i: (0, 0)),
            pl.BlockSpec((HID, HID), lambda i: (0, 0)),
            pl.BlockSpec((1, 1, BN), lambda i: (i, 0, 0)),
            pl.BlockSpec((OUT_C, HID), lambda i: (0, 0)),
            pl.BlockSpec((1, OUT_C), lambda i: (0, 0)),
        ],
        out_specs=pl.BlockSpec((G, OUT_C), lambda i: (0, 0)),
        out_shape=jax.ShapeDtypeStruct((G, OUT_C), jnp.float32),
        scratch_shapes=[
            pltpu.VMEM((G, HID), jnp.float32),
            pltpu.VMEM((1, G), jnp.float32),
        ],
    )(part1, degpart, h0, Wl1, bl1r, Wr1, batch3d, W_lin, blinr)

    return out
